# baseline (speedup 1.0000x reference)
import dataclasses
import functools

import jax
import jax.numpy as jnp
from jax import lax
from jax.experimental import pallas as pl
from jax.experimental.pallas import tpu as pltpu
from jax.experimental.pallas import tpu_sc as plsc

NC = 2
NS = 16
NW = NC * NS
LANES = 16
PACK = 4
IDX_CHUNK = 128

WIN = 512
N_CLASSES = 1000000
NWIN = 1500
CA = NWIN * WIN
ROWS_A = CA // PACK
ROWS_WIN = WIN // PACK
MAX_WPT = 2 * (-(-NWIN // (2 * NW)))


def _sc_compiler_params():
    cp = pltpu.CompilerParams(use_tc_tiling_on_sc=True)
    if "needs_layout_passes" in pltpu.CompilerParams.__dataclass_fields__:
        cp = dataclasses.replace(cp, needs_layout_passes=False)
    return cp


def _sc_relayout(centersT, d):
    wide = PACK * d
    mesh = plsc.VectorSubcoreMesh(core_axis_name="c", subcore_axis_name="s")

    @functools.partial(
        pl.kernel,
        mesh=mesh,
        compiler_params=_sc_compiler_params(),
        out_type=jax.ShapeDtypeStruct((ROWS_A, wide), jnp.float32),
        scratch_types=[
            pltpu.VMEM((d, WIN), jnp.float32),
            pltpu.VMEM((d, WIN), jnp.float32),
            pltpu.VMEM((ROWS_WIN, wide), jnp.float32),
            pltpu.VMEM((ROWS_WIN, wide), jnp.float32),
            pltpu.SemaphoreType.DMA,
            pltpu.SemaphoreType.DMA,
            pltpu.SemaphoreType.DMA,
            pltpu.SemaphoreType.DMA,
        ],
    )
    def k(ct_hbm, v_hbm, in0, in1, out0, out1, is0, is1, os0, os1):
        wid = lax.axis_index("s") * NC + lax.axis_index("c")
        inb, outb = (in0, in1), (out0, out1)
        isem, osem = (is0, is1), (os0, os1)

        f_lo = lax.iota(jnp.int32, LANES)
        f_hi = f_lo + LANES
        zero16 = jnp.zeros((LANES,), jnp.int32)

        def start_in(i, b):
            w = wid + NW * i

            @pl.when(w < NWIN)
            def _():
                pltpu.async_copy(
                    ct_hbm.at[:, pl.ds(w * WIN, WIN)], inb[b], isem[b])

        start_in(0, 0)
        start_in(1, 1)

        @pl.loop(0, MAX_WPT // 2)
        def _(p):
            for b in range(2):
                i = p * 2 + b
                w = wid + NW * i

                @pl.when(w < NWIN)
                def _():
                    @pl.when(p >= 1)
                    def _():
                        pltpu.make_async_copy(
                            outb[b], v_hbm.at[pl.ds(w * ROWS_WIN, ROWS_WIN)],
                            osem[b]).wait()
                    pltpu.make_async_copy(
                        ct_hbm.at[:, pl.ds(w * WIN, WIN)], inb[b],
                        isem[b]).wait()

                    @plsc.parallel_loop(0, ROWS_WIN, unroll=8)
                    def _(rr):
                        lbase = rr * PACK
                        for g in range(2 * PACK):
                            li = zero16 + (lbase + g // 2)
                            fi = f_lo if g % 2 == 0 else f_hi
                            outb[b][rr, pl.ds(LANES * g, LANES)] = (
                                plsc.load_gather(inb[b], [fi, li]))

                    pltpu.async_copy(
                        outb[b], v_hbm.at[pl.ds(w * ROWS_WIN, ROWS_WIN)],
                        osem[b])
                    w2 = w + 2 * NW

                    @pl.when(w2 < NWIN)
                    def _():
                        pltpu.async_copy(
                            ct_hbm.at[:, pl.ds(w2 * WIN, WIN)], inb[b],
                            isem[b])

        for b in range(2):
            pltpu.make_async_copy(
                outb[b], v_hbm.at[pl.ds(0, ROWS_WIN)], osem[b]).wait()

    return k(centersT)


def _sc_partials(label, featT, table_a, table_b, b, d):
    b_per_w = b // NW
    n_chunks = b_per_w // IDX_CHUNK
    wide = PACK * d
    mesh = plsc.VectorSubcoreMesh(core_axis_name="c", subcore_axis_name="s")

    @functools.partial(
        pl.kernel,
        mesh=mesh,
        compiler_params=_sc_compiler_params(),
        out_type=jax.ShapeDtypeStruct((NW, LANES), jnp.float32),
        scratch_types=[
            pltpu.VMEM((b_per_w,), jnp.int32),
            pltpu.VMEM((n_chunks, IDX_CHUNK), jnp.int32),
            pltpu.VMEM((n_chunks, IDX_CHUNK), jnp.int32),
            pltpu.VMEM((b_per_w,), jnp.int32),
            pltpu.VMEM((b_per_w, wide), jnp.float32),
            pltpu.VMEM((d, b_per_w), jnp.float32),
            pltpu.VMEM((LANES,), jnp.float32),
            pltpu.SemaphoreType.DMA,
            pltpu.SemaphoreType.DMA,
        ],
    )
    def k(label_hbm, featT_hbm, ta_hbm, tb_hbm, out_hbm,
          lab_v, idxa_v, idxb_v, sel_v, rows_v, featT_v, acc_v, gsem, fsem):
        wid = lax.axis_index("s") * NC + lax.axis_index("c")
        base = wid * b_per_w

        pltpu.sync_copy(label_hbm.at[pl.ds(base, b_per_w)], lab_v)
        fcp = pltpu.async_copy(
            featT_hbm.at[:, pl.ds(base, b_per_w)], featT_v, fsem)

        neg1 = jnp.full((LANES,), -1, jnp.int32)
        for kk in range(b_per_w // LANES):
            lv = lab_v[pl.ds(kk * LANES, LANES)]
            row = lax.shift_right_logical(lv, 2)
            in_a = row < ROWS_A
            sel = lax.shift_left(jnp.bitwise_and(lv, 3), 5)
            j = kk // (IDX_CHUNK // LANES)
            sl = pl.ds((kk % (IDX_CHUNK // LANES)) * LANES, LANES)
            idxa_v[j, sl] = jnp.where(in_a, row, neg1)
            idxb_v[j, sl] = jnp.where(in_a, neg1, row - ROWS_A)
            sel_v[pl.ds(kk * LANES, LANES)] = sel

        copies = []
        for j in range(n_chunks):
            copies.append(pltpu.async_copy(
                ta_hbm.at[plsc.Indices(idxa_v.at[j], ignored_value=-1)],
                rows_v.at[pl.ds(j * IDX_CHUNK, IDX_CHUNK)],
                gsem))
            copies.append(pltpu.async_copy(
                tb_hbm.at[plsc.Indices(idxb_v.at[j], ignored_value=-1)],
                rows_v.at[pl.ds(j * IDX_CHUNK, IDX_CHUNK)],
                gsem))
        fcp.wait()
        for c in copies:
            c.wait()

        lane_iota = lax.iota(jnp.int32, LANES)

        @plsc.parallel_loop(0, b_per_w // LANES, unroll=2,
                            carry=jnp.zeros((LANES,), jnp.float32))
        def acc_loop(c, acc):
            cbase = c * LANES
            row_idx = lane_iota + cbase
            col0 = sel_v[pl.ds(cbase, LANES)]
            for f in range(d):
                g = plsc.load_gather(rows_v, [row_idx, col0 + f])
                dv = featT_v[f, pl.ds(cbase, LANES)] - g
                acc = acc + dv * dv
            return acc

        acc_v[...] = acc_loop
        pltpu.sync_copy(acc_v, out_hbm.at[wid])

    return k(label, featT, table_a, table_b)


def _tc_reduce(partials, scale):
    def body(x_ref, o_ref):
        o_ref[0, 0] = jnp.sum(x_ref[...]) * scale

    return pl.pallas_call(
        body,
        out_shape=jax.ShapeDtypeStruct((1, 1), jnp.float32),
        out_specs=pl.BlockSpec(memory_space=pltpu.SMEM),
    )(partials)


def kernel(label, feat, centers):
    b, d = feat.shape
    label = label.astype(jnp.int32)
    featT = feat.T
    table_a = _sc_relayout(centers.T, d)
    table_b = centers[CA:].reshape(-1, PACK * d)
    partials = _sc_partials(label, featT, table_a, table_b, b, d)
    out = _tc_reduce(partials, 0.5 / b)
    return out.reshape(())


# speedup vs baseline: 1.1194x; 1.1194x over previous
import dataclasses
import functools

import jax
import jax.numpy as jnp
from jax import lax
from jax.experimental import pallas as pl
from jax.experimental.pallas import tpu as pltpu
from jax.experimental.pallas import tpu_sc as plsc

NC = 2
NS = 16
NW = NC * NS
LANES = 16
PACK = 4
IDX_CHUNK = 128

WIN = 512
N_CLASSES = 1000000
NWIN = 1024
CA = NWIN * WIN
ROWS_A = CA // PACK
ROWS_WIN = WIN // PACK
MAX_WPT = 2 * (-(-NWIN // (2 * NW)))


def _sc_compiler_params():
    cp = pltpu.CompilerParams(use_tc_tiling_on_sc=True)
    if "needs_layout_passes" in pltpu.CompilerParams.__dataclass_fields__:
        cp = dataclasses.replace(cp, needs_layout_passes=False)
    return cp


def _sc_relayout(centersT, d):
    wide = PACK * d
    mesh = plsc.VectorSubcoreMesh(core_axis_name="c", subcore_axis_name="s")

    @functools.partial(
        pl.kernel,
        mesh=mesh,
        compiler_params=_sc_compiler_params(),
        out_type=jax.ShapeDtypeStruct((ROWS_A, wide), jnp.float32),
        scratch_types=[
            pltpu.VMEM((d, WIN), jnp.float32),
            pltpu.VMEM((d, WIN), jnp.float32),
            pltpu.VMEM((ROWS_WIN, wide), jnp.float32),
            pltpu.VMEM((ROWS_WIN, wide), jnp.float32),
            pltpu.SemaphoreType.DMA,
            pltpu.SemaphoreType.DMA,
            pltpu.SemaphoreType.DMA,
            pltpu.SemaphoreType.DMA,
        ],
    )
    def k(ct_hbm, v_hbm, in0, in1, out0, out1, is0, is1, os0, os1):
        wid = lax.axis_index("s") * NC + lax.axis_index("c")
        inb, outb = (in0, in1), (out0, out1)
        isem, osem = (is0, is1), (os0, os1)

        f_lo = lax.iota(jnp.int32, LANES)
        f_hi = f_lo + LANES
        zero16 = jnp.zeros((LANES,), jnp.int32)

        def start_in(i, b):
            w = wid + NW * i

            @pl.when(w < NWIN)
            def _():
                pltpu.async_copy(
                    ct_hbm.at[:, pl.ds(w * WIN, WIN)], inb[b], isem[b])

        start_in(0, 0)
        start_in(1, 1)

        @pl.loop(0, MAX_WPT // 2)
        def _(p):
            for b in range(2):
                i = p * 2 + b
                w = wid + NW * i

                @pl.when(w < NWIN)
                def _():
                    @pl.when(p >= 1)
                    def _():
                        pltpu.make_async_copy(
                            outb[b], v_hbm.at[pl.ds(w * ROWS_WIN, ROWS_WIN)],
                            osem[b]).wait()
                    pltpu.make_async_copy(
                        ct_hbm.at[:, pl.ds(w * WIN, WIN)], inb[b],
                        isem[b]).wait()

                    @plsc.parallel_loop(0, ROWS_WIN, unroll=8)
                    def _(rr):
                        lbase = rr * PACK
                        for g in range(2 * PACK):
                            li = zero16 + (lbase + g // 2)
                            fi = f_lo if g % 2 == 0 else f_hi
                            outb[b][rr, pl.ds(LANES * g, LANES)] = (
                                plsc.load_gather(inb[b], [fi, li]))

                    pltpu.async_copy(
                        outb[b], v_hbm.at[pl.ds(w * ROWS_WIN, ROWS_WIN)],
                        osem[b])
                    w2 = w + 2 * NW

                    @pl.when(w2 < NWIN)
                    def _():
                        pltpu.async_copy(
                            ct_hbm.at[:, pl.ds(w2 * WIN, WIN)], inb[b],
                            isem[b])

        for b in range(2):
            pltpu.make_async_copy(
                outb[b], v_hbm.at[pl.ds(0, ROWS_WIN)], osem[b]).wait()

    return k(centersT)


def _sc_partials(label, featT, table_a, table_b, b, d):
    b_per_w = b // NW
    n_chunks = b_per_w // IDX_CHUNK
    wide = PACK * d
    mesh = plsc.VectorSubcoreMesh(core_axis_name="c", subcore_axis_name="s")

    @functools.partial(
        pl.kernel,
        mesh=mesh,
        compiler_params=_sc_compiler_params(),
        out_type=jax.ShapeDtypeStruct((NW, LANES), jnp.float32),
        scratch_types=[
            pltpu.VMEM((b_per_w,), jnp.int32),
            pltpu.VMEM((n_chunks, IDX_CHUNK), jnp.int32),
            pltpu.VMEM((n_chunks, IDX_CHUNK), jnp.int32),
            pltpu.VMEM((b_per_w,), jnp.int32),
            pltpu.VMEM((b_per_w, wide), jnp.float32),
            pltpu.VMEM((d, b_per_w), jnp.float32),
            pltpu.VMEM((LANES,), jnp.float32),
            pltpu.SemaphoreType.DMA,
            pltpu.SemaphoreType.DMA,
        ],
    )
    def k(label_hbm, featT_hbm, ta_hbm, tb_hbm, out_hbm,
          lab_v, idxa_v, idxb_v, sel_v, rows_v, featT_v, acc_v, gsem, fsem):
        wid = lax.axis_index("s") * NC + lax.axis_index("c")
        base = wid * b_per_w

        pltpu.sync_copy(label_hbm.at[pl.ds(base, b_per_w)], lab_v)
        fcp = pltpu.async_copy(
            featT_hbm.at[:, pl.ds(base, b_per_w)], featT_v, fsem)

        neg1 = jnp.full((LANES,), -1, jnp.int32)
        for kk in range(b_per_w // LANES):
            lv = lab_v[pl.ds(kk * LANES, LANES)]
            row = lax.shift_right_logical(lv, 2)
            in_a = row < ROWS_A
            sel = lax.shift_left(jnp.bitwise_and(lv, 3), 5)
            j = kk // (IDX_CHUNK // LANES)
            sl = pl.ds((kk % (IDX_CHUNK // LANES)) * LANES, LANES)
            idxa_v[j, sl] = jnp.where(in_a, row, neg1)
            idxb_v[j, sl] = jnp.where(in_a, neg1, row - ROWS_A)
            sel_v[pl.ds(kk * LANES, LANES)] = sel

        copies = []
        for j in range(n_chunks):
            copies.append(pltpu.async_copy(
                ta_hbm.at[plsc.Indices(idxa_v.at[j], ignored_value=-1)],
                rows_v.at[pl.ds(j * IDX_CHUNK, IDX_CHUNK)],
                gsem))
            copies.append(pltpu.async_copy(
                tb_hbm.at[plsc.Indices(idxb_v.at[j], ignored_value=-1)],
                rows_v.at[pl.ds(j * IDX_CHUNK, IDX_CHUNK)],
                gsem))
        fcp.wait()
        for c in copies:
            c.wait()

        lane_iota = lax.iota(jnp.int32, LANES)

        @plsc.parallel_loop(0, b_per_w // LANES, unroll=2,
                            carry=jnp.zeros((LANES,), jnp.float32))
        def acc_loop(c, acc):
            cbase = c * LANES
            row_idx = lane_iota + cbase
            col0 = sel_v[pl.ds(cbase, LANES)]
            for f in range(d):
                g = plsc.load_gather(rows_v, [row_idx, col0 + f])
                dv = featT_v[f, pl.ds(cbase, LANES)] - g
                acc = acc + dv * dv
            return acc

        acc_v[...] = acc_loop
        pltpu.sync_copy(acc_v, out_hbm.at[wid])

    return k(label, featT, table_a, table_b)


def _tc_reduce(partials, scale):
    def body(x_ref, o_ref):
        o_ref[0, 0] = jnp.sum(x_ref[...]) * scale

    return pl.pallas_call(
        body,
        out_shape=jax.ShapeDtypeStruct((1, 1), jnp.float32),
        out_specs=pl.BlockSpec(memory_space=pltpu.SMEM),
    )(partials)


def kernel(label, feat, centers):
    b, d = feat.shape
    label = label.astype(jnp.int32)
    featT = feat.T
    table_a = _sc_relayout(centers.T, d)
    table_b = centers[CA:].reshape(-1, PACK * d)
    partials = _sc_partials(label, featT, table_a, table_b, b, d)
    out = _tc_reduce(partials, 0.5 / b)
    return out.reshape(())
